# Initial kernel scaffold; baseline (speedup 1.0000x reference)
import jax
import jax.numpy as jnp
from jax.experimental import pallas as pl


def kernel(queries, targets):
    raise NotImplementedError("write your pallas kernel here")


# speedup vs baseline: 18.2301x; 18.2301x over previous
import functools

import jax
import jax.numpy as jnp
from jax.experimental import pallas as pl

_B = 1024
_D = 128
_DELTA = 1.0
_R = 128


def _body(qblk_ref, qfull_ref, tcol_ref, trow_ref, o_ref):
    b = pl.program_id(0)
    q_blk = qblk_ref[...]
    q_full = qfull_ref[...]
    tcol = tcol_ref[...]
    trow = trow_ref[...]

    n_blk = jnp.sum(q_blk * q_blk, axis=1, keepdims=True)
    ones = jnp.ones((1, _D), dtype=jnp.float32)
    n_col = jax.lax.dot_general(
        ones, q_full * q_full,
        dimension_numbers=(((1,), (1,)), ((), ())),
        preferred_element_type=jnp.float32)
    g = jax.lax.dot_general(
        q_blk, q_full,
        dimension_numbers=(((1,), (1,)), ((), ())),
        preferred_element_type=jnp.float32)
    d2 = n_blk + n_col - 2.0 * g

    row_l = jax.lax.broadcasted_iota(jnp.int32, (_R, _B), 0)
    col = jax.lax.broadcasted_iota(jnp.int32, (_R, _B), 1)
    row_g = row_l + b * _R
    not_diag = row_g != col
    same = tcol == trow

    zero = jnp.zeros((), jnp.float32)
    mp = jnp.max(jnp.where(same & not_diag, d2, zero), axis=1, keepdims=True)
    mn = jnp.max(jnp.where((~same) & not_diag, d2, zero), axis=1, keepdims=True)

    fb = jnp.where(row_g[:, 0:1] == 0, d2[0:1, 1:2], d2[:, 0:1])
    vp = jnp.where(mp > zero, mp, fb)
    vn = jnp.where(mn > zero, mn, fb)

    part = jnp.sum(jnp.maximum(_DELTA - vp + vn, zero),
                   axis=(0, 1), keepdims=True) * (1.0 / _B)

    @pl.when(b == 0)
    def _init():
        o_ref[...] = jnp.zeros_like(o_ref)

    o_ref[...] += part


@jax.jit
def _run(queries, tcol, trow):
    grid = (_B // _R,)
    return pl.pallas_call(
        _body,
        grid=grid,
        in_specs=[
            pl.BlockSpec((_R, _D), lambda b: (b, 0)),
            pl.BlockSpec((_B, _D), lambda b: (0, 0)),
            pl.BlockSpec((_R, 1), lambda b: (b, 0)),
            pl.BlockSpec((1, _B), lambda b: (0, 0)),
        ],
        out_specs=pl.BlockSpec((1, 1), lambda b: (0, 0)),
        out_shape=jax.ShapeDtypeStruct((1, 1), jnp.float32),
    )(queries, queries, tcol, trow)


def kernel(queries, targets):
    t = targets.astype(jnp.float32)
    out = _run(queries, t.reshape(_B, 1), t.reshape(1, _B))
    return out[0, 0]


# speedup vs baseline: 19.5673x; 1.0734x over previous
import functools

import jax
import jax.numpy as jnp
from jax.experimental import pallas as pl

_B = 1024
_D = 128
_DELTA = 1.0
_R = 128


def _body(qblk_ref, qfull_ref, tcol_ref, trow_ref, o_ref):
    b = pl.program_id(0)
    q_blk = qblk_ref[...]
    q_full = qfull_ref[...]
    tcol = tcol_ref[...]
    trow = trow_ref[...]

    n_blk = jnp.sum(q_blk * q_blk, axis=1, keepdims=True)
    ones = jnp.ones((1, _D), dtype=jnp.float32)
    n_col = jax.lax.dot_general(
        ones, q_full * q_full,
        dimension_numbers=(((1,), (1,)), ((), ())),
        preferred_element_type=jnp.float32)
    g2 = jax.lax.dot_general(
        -2.0 * q_blk, q_full,
        dimension_numbers=(((1,), (1,)), ((), ())),
        preferred_element_type=jnp.float32)
    d2 = (n_blk + n_col) + g2

    same = tcol == trow
    s = jnp.where(same, d2, -d2)
    zero = jnp.zeros((), jnp.float32)
    mp = jnp.maximum(jnp.max(s, axis=1, keepdims=True), zero)
    mn = jnp.maximum(-jnp.min(s, axis=1, keepdims=True), zero)

    row_g = jax.lax.broadcasted_iota(jnp.int32, (_R, 1), 0) + b * _R
    fb = jnp.where(row_g == 0, d2[0:1, 1:2], d2[:, 0:1])
    vp = jnp.where(mp > zero, mp, fb)
    vn = jnp.where(mn > zero, mn, fb)

    part = jnp.sum(jnp.maximum(_DELTA - vp + vn, zero),
                   axis=(0, 1), keepdims=True) * (1.0 / _B)

    @pl.when(b == 0)
    def _init():
        o_ref[...] = jnp.zeros_like(o_ref)

    o_ref[...] += part


@jax.jit
def _run(queries, tcol, trow):
    grid = (_B // _R,)
    return pl.pallas_call(
        _body,
        grid=grid,
        in_specs=[
            pl.BlockSpec((_R, _D), lambda b: (b, 0)),
            pl.BlockSpec((_B, _D), lambda b: (0, 0)),
            pl.BlockSpec((_R, 1), lambda b: (b, 0)),
            pl.BlockSpec((1, _B), lambda b: (0, 0)),
        ],
        out_specs=pl.BlockSpec((1, 1), lambda b: (0, 0)),
        out_shape=jax.ShapeDtypeStruct((1, 1), jnp.float32),
    )(queries, queries, tcol, trow)


def kernel(queries, targets):
    t = targets.astype(jnp.float32)
    out = _run(queries, t.reshape(_B, 1), t.reshape(1, _B))
    return out[0, 0]


# speedup vs baseline: 25.8504x; 1.3211x over previous
import functools

import jax
import jax.numpy as jnp
from jax.experimental import pallas as pl

_B = 1024
_D = 128
_DELTA = 1.0
_R = 256


def _body(qblk_ref, qfull_ref, tcol_ref, trow_ref, o_ref):
    b = pl.program_id(0)
    q_blk = qblk_ref[...]
    q_full = qfull_ref[...]
    tcol = tcol_ref[...]
    trow = trow_ref[...]

    n_blk = jnp.sum(q_blk * q_blk, axis=1, keepdims=True)
    ones = jnp.ones((1, _D), dtype=jnp.float32)
    n_col = jax.lax.dot_general(
        ones, q_full * q_full,
        dimension_numbers=(((1,), (1,)), ((), ())),
        preferred_element_type=jnp.float32)
    g2 = jax.lax.dot_general(
        -2.0 * q_blk, q_full,
        dimension_numbers=(((1,), (1,)), ((), ())),
        preferred_element_type=jnp.float32)
    d2 = (n_blk + n_col) + g2

    same = tcol == trow
    s = jnp.where(same, d2, -d2)
    zero = jnp.zeros((), jnp.float32)
    mp = jnp.maximum(jnp.max(s, axis=1, keepdims=True), zero)
    mn = jnp.maximum(-jnp.min(s, axis=1, keepdims=True), zero)

    row_g = jax.lax.broadcasted_iota(jnp.int32, (_R, 1), 0) + b * _R
    fb = jnp.where(row_g == 0, d2[0:1, 1:2], d2[:, 0:1])
    vp = jnp.where(mp > zero, mp, fb)
    vn = jnp.where(mn > zero, mn, fb)

    part = jnp.sum(jnp.maximum(_DELTA - vp + vn, zero),
                   axis=(0, 1), keepdims=True) * (1.0 / _B)

    @pl.when(b == 0)
    def _init():
        o_ref[...] = jnp.zeros_like(o_ref)

    o_ref[...] += part


@jax.jit
def _run(queries, tcol, trow):
    grid = (_B // _R,)
    return pl.pallas_call(
        _body,
        grid=grid,
        in_specs=[
            pl.BlockSpec((_R, _D), lambda b: (b, 0)),
            pl.BlockSpec((_B, _D), lambda b: (0, 0)),
            pl.BlockSpec((_R, 1), lambda b: (b, 0)),
            pl.BlockSpec((1, _B), lambda b: (0, 0)),
        ],
        out_specs=pl.BlockSpec((1, 1), lambda b: (0, 0)),
        out_shape=jax.ShapeDtypeStruct((1, 1), jnp.float32),
    )(queries, queries, tcol, trow)


def kernel(queries, targets):
    t = targets.astype(jnp.float32)
    out = _run(queries, t.reshape(_B, 1), t.reshape(1, _B))
    return out[0, 0]


# speedup vs baseline: 27.5890x; 1.0673x over previous
import functools

import jax
import jax.numpy as jnp
from jax.experimental import pallas as pl

_B = 1024
_D = 128
_DELTA = 1.0
_R = 512


def _body(qblk_ref, qfull_ref, tcol_ref, trow_ref, o_ref):
    b = pl.program_id(0)
    q_blk = qblk_ref[...]
    q_full = qfull_ref[...]
    tcol = tcol_ref[...]
    trow = trow_ref[...]

    n_blk = jnp.sum(q_blk * q_blk, axis=1, keepdims=True)
    ones = jnp.ones((1, _D), dtype=jnp.float32)
    n_col = jax.lax.dot_general(
        ones, q_full * q_full,
        dimension_numbers=(((1,), (1,)), ((), ())),
        preferred_element_type=jnp.float32)
    g2 = jax.lax.dot_general(
        -2.0 * q_blk, q_full,
        dimension_numbers=(((1,), (1,)), ((), ())),
        preferred_element_type=jnp.float32)
    d2 = (n_blk + n_col) + g2

    same = tcol == trow
    s = jnp.where(same, d2, -d2)
    zero = jnp.zeros((), jnp.float32)
    mp = jnp.maximum(jnp.max(s, axis=1, keepdims=True), zero)
    mn = jnp.maximum(-jnp.min(s, axis=1, keepdims=True), zero)

    row_g = jax.lax.broadcasted_iota(jnp.int32, (_R, 1), 0) + b * _R
    fb = jnp.where(row_g == 0, d2[0:1, 1:2], d2[:, 0:1])
    vp = jnp.where(mp > zero, mp, fb)
    vn = jnp.where(mn > zero, mn, fb)

    part = jnp.sum(jnp.maximum(_DELTA - vp + vn, zero),
                   axis=(0, 1), keepdims=True) * (1.0 / _B)

    @pl.when(b == 0)
    def _init():
        o_ref[...] = jnp.zeros_like(o_ref)

    o_ref[...] += part


@jax.jit
def _run(queries, tcol, trow):
    grid = (_B // _R,)
    return pl.pallas_call(
        _body,
        grid=grid,
        in_specs=[
            pl.BlockSpec((_R, _D), lambda b: (b, 0)),
            pl.BlockSpec((_B, _D), lambda b: (0, 0)),
            pl.BlockSpec((_R, 1), lambda b: (b, 0)),
            pl.BlockSpec((1, _B), lambda b: (0, 0)),
        ],
        out_specs=pl.BlockSpec((1, 1), lambda b: (0, 0)),
        out_shape=jax.ShapeDtypeStruct((1, 1), jnp.float32),
    )(queries, queries, tcol, trow)


def kernel(queries, targets):
    t = targets.astype(jnp.float32)
    out = _run(queries, t.reshape(_B, 1), t.reshape(1, _B))
    return out[0, 0]


# speedup vs baseline: 30.4631x; 1.1042x over previous
import functools

import jax
import jax.numpy as jnp
from jax.experimental import pallas as pl

_B = 1024
_D = 128
_DELTA = 1.0
_R = 1024


def _body(qblk_ref, qfull_ref, tcol_ref, trow_ref, o_ref):
    b = pl.program_id(0)
    q_blk = qblk_ref[...]
    q_full = qfull_ref[...]
    tcol = tcol_ref[...]
    trow = trow_ref[...]

    n_blk = jnp.sum(q_blk * q_blk, axis=1, keepdims=True)
    ones = jnp.ones((1, _D), dtype=jnp.float32)
    n_col = jax.lax.dot_general(
        ones, q_full * q_full,
        dimension_numbers=(((1,), (1,)), ((), ())),
        preferred_element_type=jnp.float32)
    g2 = jax.lax.dot_general(
        -2.0 * q_blk, q_full,
        dimension_numbers=(((1,), (1,)), ((), ())),
        preferred_element_type=jnp.float32)
    d2 = (n_blk + n_col) + g2

    same = tcol == trow
    s = jnp.where(same, d2, -d2)
    zero = jnp.zeros((), jnp.float32)
    mp = jnp.maximum(jnp.max(s, axis=1, keepdims=True), zero)
    mn = jnp.maximum(-jnp.min(s, axis=1, keepdims=True), zero)

    row_g = jax.lax.broadcasted_iota(jnp.int32, (_R, 1), 0) + b * _R
    fb = jnp.where(row_g == 0, d2[0:1, 1:2], d2[:, 0:1])
    vp = jnp.where(mp > zero, mp, fb)
    vn = jnp.where(mn > zero, mn, fb)

    part = jnp.sum(jnp.maximum(_DELTA - vp + vn, zero),
                   axis=(0, 1), keepdims=True) * (1.0 / _B)

    @pl.when(b == 0)
    def _init():
        o_ref[...] = jnp.zeros_like(o_ref)

    o_ref[...] += part


@jax.jit
def _run(queries, tcol, trow):
    grid = (_B // _R,)
    return pl.pallas_call(
        _body,
        grid=grid,
        in_specs=[
            pl.BlockSpec((_R, _D), lambda b: (b, 0)),
            pl.BlockSpec((_B, _D), lambda b: (0, 0)),
            pl.BlockSpec((_R, 1), lambda b: (b, 0)),
            pl.BlockSpec((1, _B), lambda b: (0, 0)),
        ],
        out_specs=pl.BlockSpec((1, 1), lambda b: (0, 0)),
        out_shape=jax.ShapeDtypeStruct((1, 1), jnp.float32),
    )(queries, queries, tcol, trow)


def kernel(queries, targets):
    t = targets.astype(jnp.float32)
    out = _run(queries, t.reshape(_B, 1), t.reshape(1, _B))
    return out[0, 0]


# speedup vs baseline: 31.5423x; 1.0354x over previous
import functools

import jax
import jax.numpy as jnp
from jax.experimental import pallas as pl

_B = 1024
_D = 128
_DELTA = 1.0
_R = 1024


def _body(qfull_ref, tcol_ref, trow_ref, o_ref):
    b = pl.program_id(0)
    q_full = qfull_ref[...]
    q_blk = q_full
    tcol = tcol_ref[...]
    trow = trow_ref[...]

    qq = q_full * q_full
    n_blk = jnp.sum(qq, axis=1, keepdims=True)
    ones = jnp.ones((1, _D), dtype=jnp.float32)
    n_col = jax.lax.dot_general(
        ones, qq,
        dimension_numbers=(((1,), (1,)), ((), ())),
        preferred_element_type=jnp.float32)
    g2 = jax.lax.dot_general(
        -2.0 * q_blk, q_full,
        dimension_numbers=(((1,), (1,)), ((), ())),
        preferred_element_type=jnp.float32)
    d2 = (n_blk + n_col) + g2

    same = tcol == trow
    s = jnp.where(same, d2, -d2)
    zero = jnp.zeros((), jnp.float32)
    mp = jnp.maximum(jnp.max(s, axis=1, keepdims=True), zero)
    mn = jnp.maximum(-jnp.min(s, axis=1, keepdims=True), zero)

    row_g = jax.lax.broadcasted_iota(jnp.int32, (_R, 1), 0) + b * _R
    fb = jnp.where(row_g == 0, d2[0:1, 1:2], d2[:, 0:1])
    vp = jnp.where(mp > zero, mp, fb)
    vn = jnp.where(mn > zero, mn, fb)

    part = jnp.sum(jnp.maximum(_DELTA - vp + vn, zero),
                   axis=(0, 1), keepdims=True) * (1.0 / _B)

    @pl.when(b == 0)
    def _init():
        o_ref[...] = jnp.zeros_like(o_ref)

    o_ref[...] += part


@jax.jit
def _run(queries, tcol, trow):
    grid = (_B // _R,)
    return pl.pallas_call(
        _body,
        grid=grid,
        in_specs=[
            pl.BlockSpec((_B, _D), lambda b: (0, 0)),
            pl.BlockSpec((_R, 1), lambda b: (b, 0)),
            pl.BlockSpec((1, _B), lambda b: (0, 0)),
        ],
        out_specs=pl.BlockSpec((1, 1), lambda b: (0, 0)),
        out_shape=jax.ShapeDtypeStruct((1, 1), jnp.float32),
    )(queries, tcol, trow)


def kernel(queries, targets):
    t = targets.astype(jnp.float32)
    out = _run(queries, t.reshape(_B, 1), t.reshape(1, _B))
    return out[0, 0]
